# Initial kernel scaffold; baseline (speedup 1.0000x reference)
import jax
import jax.numpy as jnp
from jax.experimental import pallas as pl


def kernel(node_features, mesh_edge_features, world_edge_features, mesh_senders, mesh_receivers, world_senders, world_receivers, mesh_W1, mesh_b1, mesh_W2, mesh_b2, world_W1, world_b1, world_W2, world_b2, node_W1, node_b1, node_W2, node_b2):
    raise NotImplementedError("write your pallas kernel here")


# speedup vs baseline: 1.0133x; 1.0133x over previous
import functools

import jax
import jax.numpy as jnp
from jax import lax
from jax.experimental import pallas as pl
from jax.experimental.pallas import tpu as pltpu
from jax.experimental.pallas import tpu_sc as plsc

N = 10000
D = 128
E_MESH = 320000
E_WORLD = 80000
CH = 128
NTILES = 32
EPM = 327680
EPW = 98304
CPM = EPM // (NTILES * CH)
CPW = EPW // (NTILES * CH)
NACC = 10240
ZROWS = NACC // 16

_f32 = jnp.float32


def _proj_body(n_ref, w_ref, t_ref):
    t_ref[...] = jnp.dot(n_ref[...], w_ref[0], preferred_element_type=_f32)


def _project(node, ws):
    blk = 1000
    return pl.pallas_call(
        _proj_body,
        grid=(4, N // blk),
        in_specs=[
            pl.BlockSpec((blk, D), lambda j, i: (i, 0)),
            pl.BlockSpec((1, D, D), lambda j, i: (j, 0, 0)),
        ],
        out_specs=pl.BlockSpec((blk, D), lambda j, i: (j * (N // blk) + i, 0)),
        out_shape=jax.ShapeDtypeStruct((4 * N, D), _f32),
    )(node, ws)


@functools.cache
def _get_sc_gather():
    mesh = plsc.VectorSubcoreMesh(
        core_axis_name="c", subcore_axis_name="s",
        num_cores=2, num_subcores=16)
    return functools.partial(
        pl.kernel,
        out_type=[jax.ShapeDtypeStruct((EPM, D), _f32),
                  jax.ShapeDtypeStruct((EPW, D), _f32)],
        mesh=mesh,
        scratch_types=[
            pltpu.VMEM((CH,), jnp.int32),
            pltpu.VMEM((CH,), jnp.int32),
            pltpu.VMEM((CH, D), _f32),
            pltpu.VMEM((CH, D), _f32),
            pltpu.SemaphoreType.DMA,
            pltpu.SemaphoreType.DMA,
        ],
    )(_sc_gather_body)


def _sc_gather_body(t_hbm, ism_hbm, irm_hbm, isw_hbm, irw_hbm, gm_hbm, gw_hbm,
                    i1_v, i2_v, rs_v, rr_v, sem1, sem2):
    wid = lax.axis_index("s") * 2 + lax.axis_index("c")

    def phase(nchunks, is_hbm, ir_hbm, dst_hbm):
        def body(k, _):
            base = (wid * nchunks + k) * CH
            pltpu.sync_copy(is_hbm.at[pl.ds(base, CH)], i1_v)
            pltpu.sync_copy(ir_hbm.at[pl.ds(base, CH)], i2_v)
            c1 = pltpu.async_copy(t_hbm.at[i1_v], rs_v, sem1)
            c2 = pltpu.async_copy(t_hbm.at[i2_v], rr_v, sem2)
            c1.wait()
            c2.wait()

            def add_row(i, _):
                for j in range(D // 16):
                    sl = (i, pl.ds(j * 16, 16))
                    rs_v[sl] = rs_v[sl] + rr_v[sl]
                return 0

            lax.fori_loop(0, CH, add_row, 0)
            pltpu.sync_copy(rs_v, dst_hbm.at[pl.ds(base, CH)])
            return 0

        lax.fori_loop(0, nchunks, body, 0)

    phase(CPM, ism_hbm, irm_hbm, gm_hbm)
    phase(CPW, isw_hbm, irw_hbm, gw_hbm)


def _edge_body(g_ref, e_ref, w1c_ref, b1_ref, w2_ref, b2_ref,
               new_ref, out_ref):
    e = e_ref[...]
    pre = (g_ref[...] + jnp.dot(e, w1c_ref[...], preferred_element_type=_f32)
           + b1_ref[...])
    h = jnp.maximum(pre, 0.0)
    new = jnp.dot(h, w2_ref[...], preferred_element_type=_f32) + b2_ref[...]
    new_ref[...] = new
    out_ref[...] = new + e


def _edge_mlp(g, ef, w1c, b1, w2, b2, e_real):
    ep = g.shape[0]
    blk = 2048
    grid = (e_real + blk - 1) // blk
    return pl.pallas_call(
        _edge_body,
        grid=(grid,),
        in_specs=[
            pl.BlockSpec((blk, D), lambda i: (i, 0)),
            pl.BlockSpec((blk, D), lambda i: (i, 0)),
            pl.BlockSpec((D, D), lambda i: (0, 0)),
            pl.BlockSpec((1, D), lambda i: (0, 0)),
            pl.BlockSpec((D, D), lambda i: (0, 0)),
            pl.BlockSpec((1, D), lambda i: (0, 0)),
        ],
        out_specs=[
            pl.BlockSpec((blk, D), lambda i: (i, 0)),
            pl.BlockSpec((blk, D), lambda i: (i, 0)),
        ],
        out_shape=[jax.ShapeDtypeStruct((ep, D), _f32),
                   jax.ShapeDtypeStruct((e_real, D), _f32)],
    )(g, ef, w1c, b1, w2, b2)


@functools.cache
def _get_sc_scatter():
    mesh = plsc.VectorSubcoreMesh(
        core_axis_name="c", subcore_axis_name="s",
        num_cores=2, num_subcores=16)
    return functools.partial(
        pl.kernel,
        out_type=[jax.ShapeDtypeStruct((2, NACC, D), _f32),
                  jax.ShapeDtypeStruct((2, NACC, D), _f32)],
        mesh=mesh,
        scratch_types=[
            pltpu.VMEM((CH,), jnp.int32),
            pltpu.VMEM((CH, D), _f32),
            pltpu.VMEM_SHARED((NACC, D), _f32),
        ],
    )(_sc_scatter_body)


def _sc_scatter_body(nm_hbm, rm_hbm, nw_hbm, rw_hbm, z_hbm, am_hbm, aw_hbm,
                     i1_v, rows_v, acc):
    c = lax.axis_index("c")
    s = lax.axis_index("s")
    wid = s * 2 + c

    def phase(nchunks, r_hbm, src_hbm, out_hbm):
        pltpu.sync_copy(z_hbm, acc.at[pl.ds(s * ZROWS, ZROWS)])
        plsc.subcore_barrier()

        def body(k, _):
            base = (wid * nchunks + k) * CH
            pltpu.sync_copy(r_hbm.at[pl.ds(base, CH)], i1_v)
            pltpu.sync_copy(src_hbm.at[pl.ds(base, CH)], rows_v)
            pltpu.sync_copy(rows_v, acc.at[i1_v], add=True)
            return 0

        lax.fori_loop(0, nchunks, body, 0)
        plsc.subcore_barrier()
        pltpu.sync_copy(acc.at[pl.ds(s * ZROWS, ZROWS)],
                        out_hbm.at[c, pl.ds(s * ZROWS, ZROWS)])
        plsc.subcore_barrier()

    phase(CPM, rm_hbm, nm_hbm, am_hbm)
    phase(CPW, rw_hbm, nw_hbm, aw_hbm)


def _node_body(n_ref, am_ref, aw_ref, w_ref, b1_ref, w2_ref, b2_ref, o_ref):
    n = n_ref[...]
    am = am_ref[0] + am_ref[1]
    aw = aw_ref[0] + aw_ref[1]
    pre = (jnp.dot(n, w_ref[0], preferred_element_type=_f32)
           + jnp.dot(am, w_ref[1], preferred_element_type=_f32)
           + jnp.dot(aw, w_ref[2], preferred_element_type=_f32)
           + b1_ref[...])
    h = jnp.maximum(pre, 0.0)
    o_ref[...] = jnp.dot(h, w2_ref[...], preferred_element_type=_f32) \
        + b2_ref[...] + n


def _node_mlp(node, am_p, aw_p, nws, b1, w2, b2):
    blk = 1000
    return pl.pallas_call(
        _node_body,
        grid=(N // blk,),
        in_specs=[
            pl.BlockSpec((blk, D), lambda i: (i, 0)),
            pl.BlockSpec((2, blk, D), lambda i: (0, i, 0)),
            pl.BlockSpec((2, blk, D), lambda i: (0, i, 0)),
            pl.BlockSpec((3, D, D), lambda i: (0, 0, 0)),
            pl.BlockSpec((1, D), lambda i: (0, 0)),
            pl.BlockSpec((D, D), lambda i: (0, 0)),
            pl.BlockSpec((1, D), lambda i: (0, 0)),
        ],
        out_specs=pl.BlockSpec((blk, D), lambda i: (i, 0)),
        out_shape=jax.ShapeDtypeStruct((N, D), _f32),
    )(node, am_p, aw_p, nws, b1, w2, b2)


def kernel(node_features, mesh_edge_features, world_edge_features,
           mesh_senders, mesh_receivers, world_senders, world_receivers,
           mesh_W1, mesh_b1, mesh_W2, mesh_b2,
           world_W1, world_b1, world_W2, world_b2,
           node_W1, node_b1, node_W2, node_b2):
    pm = EPM - E_MESH
    pw = EPW - E_WORLD
    ism = jnp.pad(mesh_senders, (0, pm))
    irm = jnp.pad(mesh_receivers + N, (0, pm))
    isw = jnp.pad(world_senders + 2 * N, (0, pw))
    irw = jnp.pad(world_receivers + 3 * N, (0, pw))
    srm = jnp.pad(mesh_receivers, (0, pm), constant_values=N)
    srw = jnp.pad(world_receivers, (0, pw), constant_values=N)
    efm = jnp.pad(mesh_edge_features, ((0, pm), (0, 0)))
    efw = jnp.pad(world_edge_features, ((0, pw), (0, 0)))
    zeros = jnp.zeros((ZROWS, D), _f32)

    ws_proj = jnp.stack([mesh_W1[:D], mesh_W1[D:2 * D],
                         world_W1[:D], world_W1[D:2 * D]])
    nws = jnp.stack([node_W1[:D], node_W1[D:2 * D], node_W1[2 * D:]])

    t = _project(node_features, ws_proj)
    gm, gw = _get_sc_gather()(t, ism, irm, isw, irw)
    new_m, out_m = _edge_mlp(gm, efm, mesh_W1[2 * D:], mesh_b1.reshape(1, D),
                             mesh_W2, mesh_b2.reshape(1, D), E_MESH)
    new_w, out_w = _edge_mlp(gw, efw, world_W1[2 * D:], world_b1.reshape(1, D),
                             world_W2, world_b2.reshape(1, D), E_WORLD)
    am_p, aw_p = _get_sc_scatter()(new_m, srm, new_w, srw, zeros)
    out_n = _node_mlp(node_features, am_p, aw_p, nws,
                      node_b1.reshape(1, D), node_W2, node_b2.reshape(1, D))
    return (out_n, out_m, out_w)


# speedup vs baseline: 1.0664x; 1.0524x over previous
import functools

import jax
import jax.numpy as jnp
from jax import lax
from jax.experimental import pallas as pl
from jax.experimental.pallas import tpu as pltpu
from jax.experimental.pallas import tpu_sc as plsc

N = 10000
D = 128
E_MESH = 320000
E_WORLD = 80000
CH = 128
NTILES = 32
EPM = 327680
EPW = 98304
CPM = EPM // (NTILES * CH)
CPW = EPW // (NTILES * CH)
NACC = 10240
ZROWS = NACC // 16

_f32 = jnp.float32


def _proj_body(n_ref, w_ref, t_ref):
    t_ref[...] = jnp.dot(n_ref[...], w_ref[0], preferred_element_type=_f32)


def _project(node, ws):
    blk = 1000
    return pl.pallas_call(
        _proj_body,
        grid=(4, N // blk),
        in_specs=[
            pl.BlockSpec((blk, D), lambda j, i: (i, 0)),
            pl.BlockSpec((1, D, D), lambda j, i: (j, 0, 0)),
        ],
        out_specs=pl.BlockSpec((blk, D), lambda j, i: (j * (N // blk) + i, 0)),
        out_shape=jax.ShapeDtypeStruct((4 * N, D), _f32),
    )(node, ws)


@functools.cache
def _get_sc_gather():
    mesh = plsc.VectorSubcoreMesh(
        core_axis_name="c", subcore_axis_name="s",
        num_cores=2, num_subcores=16)
    return functools.partial(
        pl.kernel,
        out_type=[jax.ShapeDtypeStruct((EPM, D), _f32),
                  jax.ShapeDtypeStruct((EPW, D), _f32)],
        mesh=mesh,
        scratch_types=[
            pltpu.VMEM((CPM * CH,), jnp.int32),
            pltpu.VMEM((CPM * CH,), jnp.int32),
            pltpu.VMEM((2, CH, D), _f32),
            pltpu.VMEM((2, CH, D), _f32),
            pltpu.VMEM((2, CH, D), _f32),
            pltpu.SemaphoreType.DMA,
            pltpu.SemaphoreType.DMA,
            pltpu.SemaphoreType.DMA,
            pltpu.SemaphoreType.DMA,
        ],
    )(_sc_gather_body)


def _sc_gather_body(t_hbm, ism_hbm, irm_hbm, isw_hbm, irw_hbm, gm_hbm, gw_hbm,
                    is_v, ir_v, rs_v, rr_v, og_v, sg0, sg1, so0, so1):
    wid = lax.axis_index("s") * 2 + lax.axis_index("c")

    def phase(nchunks, is_hbm, ir_hbm, dst_hbm):
        n_idx = nchunks * CH
        pltpu.sync_copy(is_hbm.at[pl.ds(wid * n_idx, n_idx)],
                        is_v.at[pl.ds(0, n_idx)])
        pltpu.sync_copy(ir_hbm.at[pl.ds(wid * n_idx, n_idx)],
                        ir_v.at[pl.ds(0, n_idx)])
        sgs = (sg0, sg1)
        sos = (so0, so1)

        def issue_gather(k, b):
            pltpu.async_copy(t_hbm.at[is_v.at[pl.ds(k * CH, CH)]],
                             rs_v.at[b], sgs[b])
            pltpu.async_copy(t_hbm.at[ir_v.at[pl.ds(k * CH, CH)]],
                             rr_v.at[b], sgs[b])

        issue_gather(0, 0)
        issue_gather(1, 1)

        def outer(g, _):
            for b in range(2):
                k = 2 * g + b
                pltpu.make_async_copy(
                    t_hbm.at[is_v.at[pl.ds(k * CH, CH)]],
                    rs_v.at[b], sgs[b]).wait()
                pltpu.make_async_copy(
                    t_hbm.at[ir_v.at[pl.ds(k * CH, CH)]],
                    rr_v.at[b], sgs[b]).wait()

                @pl.when(g >= 1)
                def _():
                    pltpu.make_async_copy(
                        og_v.at[b],
                        dst_hbm.at[pl.ds((wid * nchunks + k - 2) * CH, CH)],
                        sos[b]).wait()

                def add_row(i, _):
                    for j in range(D // 16):
                        sl = (i, pl.ds(j * 16, 16))
                        og_v[(b,) + sl] = rs_v[(b,) + sl] + rr_v[(b,) + sl]
                    return 0

                lax.fori_loop(0, CH, add_row, 0)
                pltpu.async_copy(
                    og_v.at[b],
                    dst_hbm.at[pl.ds((wid * nchunks + k) * CH, CH)], sos[b])

                @pl.when(g < nchunks // 2 - 1)
                def _():
                    issue_gather(k + 2, b)
            return 0

        lax.fori_loop(0, nchunks // 2, outer, 0)
        for b in range(2):
            pltpu.make_async_copy(
                og_v.at[b],
                dst_hbm.at[pl.ds((wid * nchunks + nchunks - 2 + b) * CH, CH)],
                sos[b]).wait()

    phase(CPM, ism_hbm, irm_hbm, gm_hbm)
    phase(CPW, isw_hbm, irw_hbm, gw_hbm)


def _edge_body(g_ref, e_ref, w1c_ref, b1_ref, w2_ref, b2_ref,
               new_ref, out_ref):
    e = e_ref[...]
    pre = (g_ref[...] + jnp.dot(e, w1c_ref[...], preferred_element_type=_f32)
           + b1_ref[...])
    h = jnp.maximum(pre, 0.0)
    new = jnp.dot(h, w2_ref[...], preferred_element_type=_f32) + b2_ref[...]
    new_ref[...] = new
    out_ref[...] = new + e


def _edge_mlp(g, ef, w1c, b1, w2, b2, e_real):
    ep = g.shape[0]
    blk = 2048
    grid = (e_real + blk - 1) // blk
    return pl.pallas_call(
        _edge_body,
        grid=(grid,),
        in_specs=[
            pl.BlockSpec((blk, D), lambda i: (i, 0)),
            pl.BlockSpec((blk, D), lambda i: (i, 0)),
            pl.BlockSpec((D, D), lambda i: (0, 0)),
            pl.BlockSpec((1, D), lambda i: (0, 0)),
            pl.BlockSpec((D, D), lambda i: (0, 0)),
            pl.BlockSpec((1, D), lambda i: (0, 0)),
        ],
        out_specs=[
            pl.BlockSpec((blk, D), lambda i: (i, 0)),
            pl.BlockSpec((blk, D), lambda i: (i, 0)),
        ],
        out_shape=[jax.ShapeDtypeStruct((ep, D), _f32),
                   jax.ShapeDtypeStruct((e_real, D), _f32)],
    )(g, ef, w1c, b1, w2, b2)


@functools.cache
def _get_sc_scatter():
    mesh = plsc.VectorSubcoreMesh(
        core_axis_name="c", subcore_axis_name="s",
        num_cores=2, num_subcores=16)
    return functools.partial(
        pl.kernel,
        out_type=[jax.ShapeDtypeStruct((2, NACC, D), _f32),
                  jax.ShapeDtypeStruct((2, NACC, D), _f32)],
        mesh=mesh,
        scratch_types=[
            pltpu.VMEM((CH,), jnp.int32),
            pltpu.VMEM((CH, D), _f32),
            pltpu.VMEM_SHARED((NACC, D), _f32),
        ],
    )(_sc_scatter_body)


def _sc_scatter_body(nm_hbm, rm_hbm, nw_hbm, rw_hbm, z_hbm, am_hbm, aw_hbm,
                     i1_v, rows_v, acc):
    c = lax.axis_index("c")
    s = lax.axis_index("s")
    wid = s * 2 + c

    def phase(nchunks, r_hbm, src_hbm, out_hbm):
        pltpu.sync_copy(z_hbm, acc.at[pl.ds(s * ZROWS, ZROWS)])
        plsc.subcore_barrier()

        def body(k, _):
            base = (wid * nchunks + k) * CH
            pltpu.sync_copy(r_hbm.at[pl.ds(base, CH)], i1_v)
            pltpu.sync_copy(src_hbm.at[pl.ds(base, CH)], rows_v)
            pltpu.sync_copy(rows_v, acc.at[i1_v], add=True)
            return 0

        lax.fori_loop(0, nchunks, body, 0)
        plsc.subcore_barrier()
        pltpu.sync_copy(acc.at[pl.ds(s * ZROWS, ZROWS)],
                        out_hbm.at[c, pl.ds(s * ZROWS, ZROWS)])
        plsc.subcore_barrier()

    phase(CPM, rm_hbm, nm_hbm, am_hbm)
    phase(CPW, rw_hbm, nw_hbm, aw_hbm)


def _node_body(n_ref, am_ref, aw_ref, w_ref, b1_ref, w2_ref, b2_ref, o_ref):
    n = n_ref[...]
    am = am_ref[0] + am_ref[1]
    aw = aw_ref[0] + aw_ref[1]
    pre = (jnp.dot(n, w_ref[0], preferred_element_type=_f32)
           + jnp.dot(am, w_ref[1], preferred_element_type=_f32)
           + jnp.dot(aw, w_ref[2], preferred_element_type=_f32)
           + b1_ref[...])
    h = jnp.maximum(pre, 0.0)
    o_ref[...] = jnp.dot(h, w2_ref[...], preferred_element_type=_f32) \
        + b2_ref[...] + n


def _node_mlp(node, am_p, aw_p, nws, b1, w2, b2):
    blk = 1000
    return pl.pallas_call(
        _node_body,
        grid=(N // blk,),
        in_specs=[
            pl.BlockSpec((blk, D), lambda i: (i, 0)),
            pl.BlockSpec((2, blk, D), lambda i: (0, i, 0)),
            pl.BlockSpec((2, blk, D), lambda i: (0, i, 0)),
            pl.BlockSpec((3, D, D), lambda i: (0, 0, 0)),
            pl.BlockSpec((1, D), lambda i: (0, 0)),
            pl.BlockSpec((D, D), lambda i: (0, 0)),
            pl.BlockSpec((1, D), lambda i: (0, 0)),
        ],
        out_specs=pl.BlockSpec((blk, D), lambda i: (i, 0)),
        out_shape=jax.ShapeDtypeStruct((N, D), _f32),
    )(node, am_p, aw_p, nws, b1, w2, b2)


def kernel(node_features, mesh_edge_features, world_edge_features,
           mesh_senders, mesh_receivers, world_senders, world_receivers,
           mesh_W1, mesh_b1, mesh_W2, mesh_b2,
           world_W1, world_b1, world_W2, world_b2,
           node_W1, node_b1, node_W2, node_b2):
    pm = EPM - E_MESH
    pw = EPW - E_WORLD
    ism = jnp.pad(mesh_senders, (0, pm))
    irm = jnp.pad(mesh_receivers + N, (0, pm))
    isw = jnp.pad(world_senders + 2 * N, (0, pw))
    irw = jnp.pad(world_receivers + 3 * N, (0, pw))
    srm = jnp.pad(mesh_receivers, (0, pm), constant_values=N)
    srw = jnp.pad(world_receivers, (0, pw), constant_values=N)
    efm = jnp.pad(mesh_edge_features, ((0, pm), (0, 0)))
    efw = jnp.pad(world_edge_features, ((0, pw), (0, 0)))
    zeros = jnp.zeros((ZROWS, D), _f32)

    ws_proj = jnp.stack([mesh_W1[:D], mesh_W1[D:2 * D],
                         world_W1[:D], world_W1[D:2 * D]])
    nws = jnp.stack([node_W1[:D], node_W1[D:2 * D], node_W1[2 * D:]])

    t = _project(node_features, ws_proj)
    gm, gw = _get_sc_gather()(t, ism, irm, isw, irw)
    new_m, out_m = _edge_mlp(gm, efm, mesh_W1[2 * D:], mesh_b1.reshape(1, D),
                             mesh_W2, mesh_b2.reshape(1, D), E_MESH)
    new_w, out_w = _edge_mlp(gw, efw, world_W1[2 * D:], world_b1.reshape(1, D),
                             world_W2, world_b2.reshape(1, D), E_WORLD)
    am_p, aw_p = _get_sc_scatter()(new_m, srm, new_w, srw, zeros)
    out_n = _node_mlp(node_features, am_p, aw_p, nws,
                      node_b1.reshape(1, D), node_W2, node_b2.reshape(1, D))
    return (out_n, out_m, out_w)
